# baseline (speedup 1.0000x reference)
import functools
import math

import jax
import jax.numpy as jnp
from jax import lax
from jax.experimental import pallas as pl
from jax.experimental.pallas import tpu as pltpu
from jax.experimental.pallas import tpu_sc as plsc

_NC = 2
_NS = 16
_NW = _NC * _NS
_LANES = 16
_WIN = 256

_LAMDA = 0.5


_NBUF = 3


@functools.lru_cache(maxsize=None)
def _make_prop(n, wpw, h):
    assert wpw % _NBUF == 0 and wpw >= 2 * _NBUF
    rps = (-(-n // _NS) + 7) // 8 * 8
    npad = rps * _NS
    ncol = h // _LANES
    assert n % _NS == 0
    hps = n // _NS

    mesh = plsc.VectorSubcoreMesh(core_axis_name="c", subcore_axis_name="s")

    def body(h_hbm, src_hbm, dst_hbm, norm_hbm, zer_hbm, out_hbm,
             acc, sbuf, dbuf, nbuf, rows,
             sg0, sr0, sr1, sr2, sr3, ss0, ss1, ss2, ss3):
        cid = lax.axis_index("c")
        sid = lax.axis_index("s")
        wid = sid * _NC + cid
        row0 = wid * wpw
        sr = (sr0, sr1, sr2, sr3)
        ss = (ss0, ss1, ss2, ss3)

        def issue_gather(k, p):
            pltpu.async_copy(h_hbm.at[sbuf.at[k]], rows.at[p], sr[p])

        def wait_gather(p):
            pltpu.make_async_copy(h_hbm.at[pl.ds(0, _WIN)], rows.at[p], sr[p]).wait()

        def issue_scatter(k, p):
            pltpu.async_copy(rows.at[p], acc.at[dbuf.at[k]], ss[p], add=True)

        def wait_scatter(p):
            pltpu.make_async_copy(h_hbm.at[pl.ds(0, _WIN)], rows.at[p], ss[p]).wait()

        cp_s = pltpu.async_copy(src_hbm.at[pl.ds(row0, wpw)], sbuf.at[pl.ds(0, wpw)], sg0)
        cp_d = pltpu.async_copy(dst_hbm.at[pl.ds(row0, wpw)], dbuf.at[pl.ds(0, wpw)], sg0)
        cp_n = pltpu.async_copy(norm_hbm.at[pl.ds(row0, wpw)], nbuf, sg0)

        zf = jnp.zeros((_LANES,), jnp.float32)

        def zrow(r, c):
            for cc in range(ncol):
                rows[_NBUF - 1, r, pl.ds(cc * _LANES, _LANES)] = zf
            return c
        lax.fori_loop(0, _WIN, zrow, 0)
        for cc in range(_WIN // _LANES):
            dbuf[wpw, pl.ds(cc * _LANES, _LANES)] = (
                lax.iota(jnp.int32, _LANES) + cc * _LANES)

        pltpu.sync_copy(zer_hbm, acc.at[pl.ds(sid * rps, rps)])
        plsc.subcore_barrier()

        pltpu.async_copy(rows.at[_NBUF - 1], acc.at[dbuf.at[wpw]], ss[_NBUF - 1],
                         add=True)

        cp_s.wait()
        cp_d.wait()
        cp_n.wait()
        for k in range(_NBUF - 1):
            issue_gather(k, k)

        def scale(k, p):
            def gbody(g16, c2):
                nv16 = nbuf[k, pl.ds(g16 * _LANES, _LANES)]
                for l in range(_LANES):
                    vb = jnp.full((_LANES,), nv16[l], jnp.float32)
                    ei = g16 * _LANES + l
                    for cc in range(ncol):
                        sl = pl.ds(cc * _LANES, _LANES)
                        rows[p, ei, sl] = rows[p, ei, sl] * vb
                return c2
            lax.fori_loop(0, _WIN // _LANES, gbody, 0)

        def window(k4, sub):
            k = _NBUF * k4 + sub
            p = sub
            q = (sub + _NBUF - 1) % _NBUF
            wait_gather(p)

            def ahead():
                wait_scatter(q)
                issue_gather(k + _NBUF - 1, q)

            if sub == 0:
                ahead()
            else:
                pl.when(k4 < wpw // _NBUF - 1)(ahead)

            scale(k, p)
            issue_scatter(k, p)

        def k4body(k4, c):
            for sub in range(_NBUF):
                window(k4, sub)
            return c
        lax.fori_loop(0, wpw // _NBUF, k4body, 0)

        for p in range(_NBUF):
            wait_scatter(p)
        plsc.subcore_barrier()
        pltpu.sync_copy(acc.at[pl.ds(sid * rps, rps)],
                        out_hbm.at[cid, pl.ds(sid * rps, rps)])

    return pl.kernel(
        body,
        out_type=jax.ShapeDtypeStruct((2, npad, h), jnp.float32),
        mesh=mesh,
        compiler_params=pltpu.CompilerParams(use_tc_tiling_on_sc=False),
        scratch_types=[
            pltpu.VMEM_SHARED((npad, h), jnp.float32),
            pltpu.VMEM((wpw, _WIN), jnp.int32),
            pltpu.VMEM((wpw + 8, _WIN), jnp.int32),
            pltpu.VMEM((wpw, _WIN), jnp.float32),
            pltpu.VMEM((_NBUF, _WIN, h), jnp.float32),
            pltpu.SemaphoreType.DMA,
            pltpu.SemaphoreType.DMA,
            pltpu.SemaphoreType.DMA,
            pltpu.SemaphoreType.DMA,
            pltpu.SemaphoreType.DMA,
            pltpu.SemaphoreType.DMA,
            pltpu.SemaphoreType.DMA,
            pltpu.SemaphoreType.DMA,
            pltpu.SemaphoreType.DMA,
        ],
    )


_BLK = 400


def _fc0(features, w, b):
    n, din = features.shape
    hdim = w.shape[1]

    def bdy(x_ref, w_ref, b_ref, o_ref):
        o_ref[...] = jnp.maximum(
            jnp.dot(x_ref[...], w_ref[...], preferred_element_type=jnp.float32)
            + b_ref[...], 0.0)

    return pl.pallas_call(
        bdy,
        grid=(n // _BLK,),
        in_specs=[
            pl.BlockSpec((_BLK, din), lambda i: (i, 0)),
            pl.BlockSpec((din, hdim), lambda i: (0, 0)),
            pl.BlockSpec((1, hdim), lambda i: (0, 0)),
        ],
        out_specs=pl.BlockSpec((_BLK, hdim), lambda i: (i, 0)),
        out_shape=jax.ShapeDtypeStruct((n, hdim), jnp.float32),
    )(features, w, b.reshape(1, hdim))


def _layer(a, h0, pp, prev, w, b, *, beta, tmul, pmul, dorelu):
    n, hdim = h0.shape

    def bdy(a_ref, h0_ref, p0_ref, p1_ref, pv_ref, w_ref, b_ref, o_ref):
        av = a_ref[0]
        tx = tmul * (p0_ref[0] + p1_ref[0]) - pmul * pv_ref[...]
        hi = av * h0_ref[...] + (1.0 - av) * tx
        x = ((1.0 - beta) * hi
             + beta * jnp.dot(hi, w_ref[...], preferred_element_type=jnp.float32)
             + b_ref[...])
        o_ref[...] = jnp.maximum(x, 0.0) if dorelu else x

    return pl.pallas_call(
        bdy,
        grid=(n // _BLK,),
        in_specs=[
            pl.BlockSpec(memory_space=pltpu.SMEM),
            pl.BlockSpec((_BLK, hdim), lambda i: (i, 0)),
            pl.BlockSpec((1, _BLK, hdim), lambda i: (0, i, 0)),
            pl.BlockSpec((1, _BLK, hdim), lambda i: (1, i, 0)),
            pl.BlockSpec((_BLK, hdim), lambda i: (i, 0)),
            pl.BlockSpec((hdim, hdim), lambda i: (0, 0)),
            pl.BlockSpec((1, hdim), lambda i: (0, 0)),
        ],
        out_specs=pl.BlockSpec((_BLK, hdim), lambda i: (i, 0)),
        out_shape=jax.ShapeDtypeStruct((n, hdim), jnp.float32),
    )(a, h0, pp, pp, prev, w, b.reshape(1, hdim))


def _layer0(h0, w, b, *, beta):
    n, hdim = h0.shape

    def bdy(h0_ref, w_ref, b_ref, o_ref):
        hi = h0_ref[...]
        x = ((1.0 - beta) * hi
             + beta * jnp.dot(hi, w_ref[...], preferred_element_type=jnp.float32)
             + b_ref[...])
        o_ref[...] = jnp.maximum(x, 0.0)

    return pl.pallas_call(
        bdy,
        grid=(n // _BLK,),
        in_specs=[
            pl.BlockSpec((_BLK, hdim), lambda i: (i, 0)),
            pl.BlockSpec((hdim, hdim), lambda i: (0, 0)),
            pl.BlockSpec((1, hdim), lambda i: (0, 0)),
        ],
        out_specs=pl.BlockSpec((_BLK, hdim), lambda i: (i, 0)),
        out_shape=jax.ShapeDtypeStruct((n, hdim), jnp.float32),
    )(h0, w, b.reshape(1, hdim))


def _final(x, w, b):
    n, hdim = x.shape
    c = w.shape[1]

    def bdy(x_ref, w_ref, b_ref, o_ref):
        t = jnp.maximum(x_ref[...], 0.0)
        y = (jnp.dot(t, w_ref[...], preferred_element_type=jnp.float32)
             + b_ref[...])
        m = jnp.max(y, axis=1, keepdims=True)
        lse = m + jnp.log(jnp.sum(jnp.exp(y - m), axis=1, keepdims=True))
        o_ref[...] = y - lse

    return pl.pallas_call(
        bdy,
        grid=(n // _BLK,),
        in_specs=[
            pl.BlockSpec((_BLK, hdim), lambda i: (i, 0)),
            pl.BlockSpec((hdim, c), lambda i: (0, 0)),
            pl.BlockSpec((1, c), lambda i: (0, 0)),
        ],
        out_specs=pl.BlockSpec((_BLK, c), lambda i: (i, 0)),
        out_shape=jax.ShapeDtypeStruct((n, c), jnp.float32),
    )(x, w, b.reshape(1, c))


def kernel(features, edge_index, norm_A, W_fc0, b_fc0, conv_W, conv_b,
           W_fc1, b_fc1, alpha_params):
    n = features.shape[0]
    e = norm_A.shape[0]
    hdim = W_fc0.shape[1]
    lnum = conv_W.shape[0] - 1

    wpw = -(-e // (_WIN * _NW))
    wpw = (wpw + _NBUF - 1) // _NBUF * _NBUF
    wpw = max(wpw, 2 * _NBUF)
    epad = wpw * _NW * _WIN
    pad = epad - e
    src2 = jnp.pad(edge_index[0], (0, pad)).reshape(epad // _WIN, _WIN)
    dst2 = jnp.pad(edge_index[1], (0, pad)).reshape(epad // _WIN, _WIN)
    norm2 = jnp.pad(norm_A, (0, pad)).reshape(epad // _WIN, _WIN)
    zer = jnp.zeros(((-(-n // _NS) + 7) // 8 * 8, hdim), jnp.float32)
    prop = _make_prop(n, wpw, hdim)

    h0 = _fc0(features, W_fc0, b_fc0)
    x = _layer0(h0, conv_W[0], conv_b[0],
                beta=math.log(_LAMDA / 1.0 + 1.0))
    prev = h0
    last = x
    for i in range(1, lnum + 1):
        pp = prop(last, src2, dst2, norm2, zer)
        a = alpha_params[lnum - i].reshape(1)
        beta = math.log(_LAMDA / (i + 1) + 1.0)
        xi = _layer(a, h0, pp, prev, conv_W[i], conv_b[i],
                    beta=beta, tmul=1.0 if i == 1 else 2.0,
                    pmul=0.0 if i == 1 else 1.0,
                    dorelu=i < lnum - 1)
        prev = last
        last = xi
    return _final(last, W_fc1, b_fc1)


# speedup vs baseline: 1.9216x; 1.9216x over previous
import functools
import math

import jax
import jax.numpy as jnp
from jax import lax
from jax.experimental import pallas as pl
from jax.experimental.pallas import tpu as pltpu
from jax.experimental.pallas import tpu_sc as plsc

_NC = 2
_NS = 16
_NW = _NC * _NS
_LANES = 16
_WIN = 128

_LAMDA = 0.5


_NBUF = 2


@functools.lru_cache(maxsize=None)
def _make_prop(n, wpw, h):
    assert wpw % _NBUF == 0 and wpw >= 2 * _NBUF
    rps = (-(-n // _NS) + 7) // 8 * 8
    npad = rps * _NS
    ncol = h // _LANES
    assert n % _NS == 0
    hps = n // _NS

    mesh = plsc.VectorSubcoreMesh(core_axis_name="c", subcore_axis_name="s")

    def body(h_hbm, src_hbm, dst_hbm, norm_hbm, zer_hbm, out_hbm,
             acc, hsp, sbuf, dbuf, nbuf, rows,
             sg0, sr0, sr1, sr2, sr3, ss0, ss1, ss2, ss3):
        cid = lax.axis_index("c")
        sid = lax.axis_index("s")
        wid = sid * _NC + cid
        row0 = wid * wpw
        sr = (sr0, sr1, sr2, sr3)
        ss = (ss0, ss1, ss2, ss3)

        def issue_gather(k, p):
            src = h_hbm if p % 2 == 0 else hsp
            pltpu.async_copy(src.at[sbuf.at[k]], rows.at[p], sr[p])

        def wait_gather(p):
            src = h_hbm if p % 2 == 0 else hsp
            pltpu.make_async_copy(src.at[pl.ds(0, _WIN)], rows.at[p], sr[p]).wait()

        def issue_scatter(k, p):
            pltpu.async_copy(rows.at[p], acc.at[dbuf.at[k]], ss[p], add=True)

        def wait_scatter(p):
            pltpu.make_async_copy(h_hbm.at[pl.ds(0, _WIN)], rows.at[p], ss[p]).wait()

        cp_s = pltpu.async_copy(src_hbm.at[pl.ds(row0, wpw)], sbuf.at[pl.ds(0, wpw)], sg0)
        cp_d = pltpu.async_copy(dst_hbm.at[pl.ds(row0, wpw)], dbuf.at[pl.ds(0, wpw)], sg0)
        cp_n = pltpu.async_copy(norm_hbm.at[pl.ds(row0, wpw)], nbuf, sg0)

        zf = jnp.zeros((_LANES,), jnp.float32)

        def zrow(r, c):
            for cc in range(ncol):
                rows[_NBUF - 1, r, pl.ds(cc * _LANES, _LANES)] = zf
            return c
        lax.fori_loop(0, _WIN, zrow, 0)
        for cc in range(_WIN // _LANES):
            dbuf[wpw, pl.ds(cc * _LANES, _LANES)] = (
                lax.iota(jnp.int32, _LANES) + cc * _LANES)

        pltpu.sync_copy(zer_hbm, acc.at[pl.ds(sid * rps, rps)])
        pltpu.sync_copy(h_hbm.at[pl.ds(sid * hps, hps)],
                        hsp.at[pl.ds(sid * hps, hps)])
        plsc.subcore_barrier()

        pltpu.async_copy(rows.at[_NBUF - 1], acc.at[dbuf.at[wpw]], ss[_NBUF - 1],
                         add=True)

        cp_s.wait()
        cp_d.wait()
        cp_n.wait()
        for k in range(_NBUF - 1):
            issue_gather(k, k)

        def scale(k, p):
            def gbody(g16, c2):
                nv16 = nbuf[k, pl.ds(g16 * _LANES, _LANES)]
                for l in range(_LANES):
                    vb = jnp.full((_LANES,), nv16[l], jnp.float32)
                    ei = g16 * _LANES + l
                    for cc in range(ncol):
                        sl = pl.ds(cc * _LANES, _LANES)
                        rows[p, ei, sl] = rows[p, ei, sl] * vb
                return c2
            lax.fori_loop(0, _WIN // _LANES, gbody, 0)

        def window(k4, sub):
            k = _NBUF * k4 + sub
            p = sub
            q = (sub + _NBUF - 1) % _NBUF
            wait_gather(p)

            def ahead():
                wait_scatter(q)
                issue_gather(k + _NBUF - 1, q)

            if sub == 0:
                ahead()
            else:
                pl.when(k4 < wpw // _NBUF - 1)(ahead)

            scale(k, p)
            issue_scatter(k, p)

        def k4body(k4, c):
            for sub in range(_NBUF):
                window(k4, sub)
            return c
        lax.fori_loop(0, wpw // _NBUF, k4body, 0)

        for p in range(_NBUF):
            wait_scatter(p)
        plsc.subcore_barrier()
        pltpu.sync_copy(acc.at[pl.ds(sid * rps, rps)],
                        out_hbm.at[cid, pl.ds(sid * rps, rps)])

    return pl.kernel(
        body,
        out_type=jax.ShapeDtypeStruct((2, npad, h), jnp.float32),
        mesh=mesh,
        compiler_params=pltpu.CompilerParams(use_tc_tiling_on_sc=False),
        scratch_types=[
            pltpu.VMEM_SHARED((npad, h), jnp.float32),
            pltpu.VMEM_SHARED((n, h), jnp.float32),
            pltpu.VMEM((wpw, _WIN), jnp.int32),
            pltpu.VMEM((wpw + 8, _WIN), jnp.int32),
            pltpu.VMEM((wpw, _WIN), jnp.float32),
            pltpu.VMEM((_NBUF, _WIN, h), jnp.float32),
            pltpu.SemaphoreType.DMA,
            pltpu.SemaphoreType.DMA,
            pltpu.SemaphoreType.DMA,
            pltpu.SemaphoreType.DMA,
            pltpu.SemaphoreType.DMA,
            pltpu.SemaphoreType.DMA,
            pltpu.SemaphoreType.DMA,
            pltpu.SemaphoreType.DMA,
            pltpu.SemaphoreType.DMA,
        ],
    )


_BLK = 400


def _fc0(features, w, b):
    n, din = features.shape
    hdim = w.shape[1]

    def bdy(x_ref, w_ref, b_ref, o_ref):
        o_ref[...] = jnp.maximum(
            jnp.dot(x_ref[...], w_ref[...], preferred_element_type=jnp.float32)
            + b_ref[...], 0.0)

    return pl.pallas_call(
        bdy,
        grid=(n // _BLK,),
        in_specs=[
            pl.BlockSpec((_BLK, din), lambda i: (i, 0)),
            pl.BlockSpec((din, hdim), lambda i: (0, 0)),
            pl.BlockSpec((1, hdim), lambda i: (0, 0)),
        ],
        out_specs=pl.BlockSpec((_BLK, hdim), lambda i: (i, 0)),
        out_shape=jax.ShapeDtypeStruct((n, hdim), jnp.float32),
    )(features, w, b.reshape(1, hdim))


def _layer(a, h0, pp, prev, w, b, *, beta, tmul, pmul, dorelu):
    n, hdim = h0.shape

    def bdy(a_ref, h0_ref, p0_ref, p1_ref, pv_ref, w_ref, b_ref, o_ref):
        av = a_ref[0]
        tx = tmul * (p0_ref[0] + p1_ref[0]) - pmul * pv_ref[...]
        hi = av * h0_ref[...] + (1.0 - av) * tx
        x = ((1.0 - beta) * hi
             + beta * jnp.dot(hi, w_ref[...], preferred_element_type=jnp.float32)
             + b_ref[...])
        o_ref[...] = jnp.maximum(x, 0.0) if dorelu else x

    return pl.pallas_call(
        bdy,
        grid=(n // _BLK,),
        in_specs=[
            pl.BlockSpec(memory_space=pltpu.SMEM),
            pl.BlockSpec((_BLK, hdim), lambda i: (i, 0)),
            pl.BlockSpec((1, _BLK, hdim), lambda i: (0, i, 0)),
            pl.BlockSpec((1, _BLK, hdim), lambda i: (1, i, 0)),
            pl.BlockSpec((_BLK, hdim), lambda i: (i, 0)),
            pl.BlockSpec((hdim, hdim), lambda i: (0, 0)),
            pl.BlockSpec((1, hdim), lambda i: (0, 0)),
        ],
        out_specs=pl.BlockSpec((_BLK, hdim), lambda i: (i, 0)),
        out_shape=jax.ShapeDtypeStruct((n, hdim), jnp.float32),
    )(a, h0, pp, pp, prev, w, b.reshape(1, hdim))


def _layer0(h0, w, b, *, beta):
    n, hdim = h0.shape

    def bdy(h0_ref, w_ref, b_ref, o_ref):
        hi = h0_ref[...]
        x = ((1.0 - beta) * hi
             + beta * jnp.dot(hi, w_ref[...], preferred_element_type=jnp.float32)
             + b_ref[...])
        o_ref[...] = jnp.maximum(x, 0.0)

    return pl.pallas_call(
        bdy,
        grid=(n // _BLK,),
        in_specs=[
            pl.BlockSpec((_BLK, hdim), lambda i: (i, 0)),
            pl.BlockSpec((hdim, hdim), lambda i: (0, 0)),
            pl.BlockSpec((1, hdim), lambda i: (0, 0)),
        ],
        out_specs=pl.BlockSpec((_BLK, hdim), lambda i: (i, 0)),
        out_shape=jax.ShapeDtypeStruct((n, hdim), jnp.float32),
    )(h0, w, b.reshape(1, hdim))


def _final(x, w, b):
    n, hdim = x.shape
    c = w.shape[1]

    def bdy(x_ref, w_ref, b_ref, o_ref):
        t = jnp.maximum(x_ref[...], 0.0)
        y = (jnp.dot(t, w_ref[...], preferred_element_type=jnp.float32)
             + b_ref[...])
        m = jnp.max(y, axis=1, keepdims=True)
        lse = m + jnp.log(jnp.sum(jnp.exp(y - m), axis=1, keepdims=True))
        o_ref[...] = y - lse

    return pl.pallas_call(
        bdy,
        grid=(n // _BLK,),
        in_specs=[
            pl.BlockSpec((_BLK, hdim), lambda i: (i, 0)),
            pl.BlockSpec((hdim, c), lambda i: (0, 0)),
            pl.BlockSpec((1, c), lambda i: (0, 0)),
        ],
        out_specs=pl.BlockSpec((_BLK, c), lambda i: (i, 0)),
        out_shape=jax.ShapeDtypeStruct((n, c), jnp.float32),
    )(x, w, b.reshape(1, c))


def kernel(features, edge_index, norm_A, W_fc0, b_fc0, conv_W, conv_b,
           W_fc1, b_fc1, alpha_params):
    n = features.shape[0]
    e = norm_A.shape[0]
    hdim = W_fc0.shape[1]
    lnum = conv_W.shape[0] - 1

    wpw = -(-e // (_WIN * _NW))
    wpw = (wpw + _NBUF - 1) // _NBUF * _NBUF
    wpw = max(wpw, 2 * _NBUF)
    epad = wpw * _NW * _WIN
    pad = epad - e
    src2 = jnp.pad(edge_index[0], (0, pad)).reshape(epad // _WIN, _WIN)
    dst2 = jnp.pad(edge_index[1], (0, pad)).reshape(epad // _WIN, _WIN)
    norm2 = jnp.pad(norm_A, (0, pad)).reshape(epad // _WIN, _WIN)
    zer = jnp.zeros(((-(-n // _NS) + 7) // 8 * 8, hdim), jnp.float32)
    prop = _make_prop(n, wpw, hdim)

    h0 = _fc0(features, W_fc0, b_fc0)
    x = _layer0(h0, conv_W[0], conv_b[0],
                beta=math.log(_LAMDA / 1.0 + 1.0))
    prev = h0
    last = x
    for i in range(1, lnum + 1):
        pp = prop(last, src2, dst2, norm2, zer)
        a = alpha_params[lnum - i].reshape(1)
        beta = math.log(_LAMDA / (i + 1) + 1.0)
        xi = _layer(a, h0, pp, prev, conv_W[i], conv_b[i],
                    beta=beta, tmul=1.0 if i == 1 else 2.0,
                    pmul=0.0 if i == 1 else 1.0,
                    dorelu=i < lnum - 1)
        prev = last
        last = xi
    return _final(last, W_fc1, b_fc1)


# speedup vs baseline: 2.2364x; 1.1638x over previous
import functools
import math

import jax
import jax.numpy as jnp
from jax import lax
from jax.experimental import pallas as pl
from jax.experimental.pallas import tpu as pltpu
from jax.experimental.pallas import tpu_sc as plsc

_NC = 2
_NS = 16
_NW = _NC * _NS
_LANES = 16
_WIN = 128

_LAMDA = 0.5


_NBUF = 2


@functools.lru_cache(maxsize=None)
def _make_prop(n, wpw, h):
    assert wpw % _NBUF == 0 and wpw >= 2 * _NBUF
    rps = (-(-n // _NS) + 7) // 8 * 8
    npad = rps * _NS
    ncol = h // _LANES
    assert n % _NS == 0
    hps = n // _NS

    mesh = plsc.VectorSubcoreMesh(core_axis_name="c", subcore_axis_name="s")

    def body(h_hbm, src_hbm, dst_hbm, norm_hbm, zer_hbm, out_hbm,
             acc, hsp, sbuf, dbuf, nbuf, rows,
             sg0, sr0, sr1, sr2, sr3, ss0, ss1, ss2, ss3):
        cid = lax.axis_index("c")
        sid = lax.axis_index("s")
        wid = sid * _NC + cid
        row0 = wid * wpw
        sr = (sr0, sr1, sr2, sr3)
        ss = (ss0, ss1, ss2, ss3)

        def issue_gather(k, p):
            pltpu.async_copy(hsp.at[sbuf.at[k]], rows.at[p], sr[p])

        def wait_gather(p):
            pltpu.make_async_copy(hsp.at[pl.ds(0, _WIN)], rows.at[p], sr[p]).wait()

        def issue_scatter(k, p):
            pltpu.async_copy(rows.at[p], acc.at[dbuf.at[k]], ss[p], add=True)

        def wait_scatter(p):
            pltpu.make_async_copy(h_hbm.at[pl.ds(0, _WIN)], rows.at[p], ss[p]).wait()

        cp_s = pltpu.async_copy(src_hbm.at[pl.ds(row0, wpw)], sbuf.at[pl.ds(0, wpw)], sg0)
        cp_d = pltpu.async_copy(dst_hbm.at[pl.ds(row0, wpw)], dbuf.at[pl.ds(0, wpw)], sg0)
        cp_n = pltpu.async_copy(norm_hbm.at[pl.ds(row0, wpw)], nbuf, sg0)

        zf = jnp.zeros((_LANES,), jnp.float32)

        def zrow(r, c):
            for cc in range(ncol):
                rows[_NBUF - 1, r, pl.ds(cc * _LANES, _LANES)] = zf
            return c
        lax.fori_loop(0, _WIN, zrow, 0)
        for cc in range(_WIN // _LANES):
            dbuf[wpw, pl.ds(cc * _LANES, _LANES)] = (
                lax.iota(jnp.int32, _LANES) + cc * _LANES)

        pltpu.sync_copy(zer_hbm, acc.at[pl.ds(sid * rps, rps)])
        pltpu.sync_copy(h_hbm.at[pl.ds(sid * hps, hps)],
                        hsp.at[pl.ds(sid * hps, hps)])
        plsc.subcore_barrier()

        pltpu.async_copy(rows.at[_NBUF - 1], acc.at[dbuf.at[wpw]], ss[_NBUF - 1],
                         add=True)

        cp_s.wait()
        cp_d.wait()
        cp_n.wait()
        for k in range(_NBUF - 1):
            issue_gather(k, k)

        def scale(k, p):
            def gbody(g16, c2):
                nv16 = nbuf[k, pl.ds(g16 * _LANES, _LANES)]
                for l in range(_LANES):
                    vb = jnp.full((_LANES,), nv16[l], jnp.float32)
                    ei = g16 * _LANES + l
                    for cc in range(ncol):
                        sl = pl.ds(cc * _LANES, _LANES)
                        rows[p, ei, sl] = rows[p, ei, sl] * vb
                return c2
            lax.fori_loop(0, _WIN // _LANES, gbody, 0)

        def window(k4, sub):
            k = _NBUF * k4 + sub
            p = sub
            q = (sub + _NBUF - 1) % _NBUF
            wait_gather(p)

            def ahead():
                wait_scatter(q)
                issue_gather(k + _NBUF - 1, q)

            if sub == 0:
                ahead()
            else:
                pl.when(k4 < wpw // _NBUF - 1)(ahead)

            scale(k, p)
            issue_scatter(k, p)

        def k4body(k4, c):
            for sub in range(_NBUF):
                window(k4, sub)
            return c
        lax.fori_loop(0, wpw // _NBUF, k4body, 0)

        for p in range(_NBUF):
            wait_scatter(p)
        plsc.subcore_barrier()
        pltpu.sync_copy(acc.at[pl.ds(sid * rps, rps)],
                        out_hbm.at[cid, pl.ds(sid * rps, rps)])

    return pl.kernel(
        body,
        out_type=jax.ShapeDtypeStruct((2, npad, h), jnp.float32),
        mesh=mesh,
        compiler_params=pltpu.CompilerParams(use_tc_tiling_on_sc=False),
        scratch_types=[
            pltpu.VMEM_SHARED((npad, h), jnp.float32),
            pltpu.VMEM_SHARED((n, h), jnp.float32),
            pltpu.VMEM((wpw, _WIN), jnp.int32),
            pltpu.VMEM((wpw + 8, _WIN), jnp.int32),
            pltpu.VMEM((wpw, _WIN), jnp.float32),
            pltpu.VMEM((_NBUF, _WIN, h), jnp.float32),
            pltpu.SemaphoreType.DMA,
            pltpu.SemaphoreType.DMA,
            pltpu.SemaphoreType.DMA,
            pltpu.SemaphoreType.DMA,
            pltpu.SemaphoreType.DMA,
            pltpu.SemaphoreType.DMA,
            pltpu.SemaphoreType.DMA,
            pltpu.SemaphoreType.DMA,
            pltpu.SemaphoreType.DMA,
        ],
    )


_BLK = 400


def _fc0(features, w, b):
    n, din = features.shape
    hdim = w.shape[1]

    def bdy(x_ref, w_ref, b_ref, o_ref):
        o_ref[...] = jnp.maximum(
            jnp.dot(x_ref[...], w_ref[...], preferred_element_type=jnp.float32)
            + b_ref[...], 0.0)

    return pl.pallas_call(
        bdy,
        grid=(n // _BLK,),
        in_specs=[
            pl.BlockSpec((_BLK, din), lambda i: (i, 0)),
            pl.BlockSpec((din, hdim), lambda i: (0, 0)),
            pl.BlockSpec((1, hdim), lambda i: (0, 0)),
        ],
        out_specs=pl.BlockSpec((_BLK, hdim), lambda i: (i, 0)),
        out_shape=jax.ShapeDtypeStruct((n, hdim), jnp.float32),
    )(features, w, b.reshape(1, hdim))


def _layer(a, h0, pp, prev, w, b, *, beta, tmul, pmul, dorelu):
    n, hdim = h0.shape

    def bdy(a_ref, h0_ref, p0_ref, p1_ref, pv_ref, w_ref, b_ref, o_ref):
        av = a_ref[0]
        tx = tmul * (p0_ref[0] + p1_ref[0]) - pmul * pv_ref[...]
        hi = av * h0_ref[...] + (1.0 - av) * tx
        x = ((1.0 - beta) * hi
             + beta * jnp.dot(hi, w_ref[...], preferred_element_type=jnp.float32)
             + b_ref[...])
        o_ref[...] = jnp.maximum(x, 0.0) if dorelu else x

    return pl.pallas_call(
        bdy,
        grid=(n // _BLK,),
        in_specs=[
            pl.BlockSpec(memory_space=pltpu.SMEM),
            pl.BlockSpec((_BLK, hdim), lambda i: (i, 0)),
            pl.BlockSpec((1, _BLK, hdim), lambda i: (0, i, 0)),
            pl.BlockSpec((1, _BLK, hdim), lambda i: (1, i, 0)),
            pl.BlockSpec((_BLK, hdim), lambda i: (i, 0)),
            pl.BlockSpec((hdim, hdim), lambda i: (0, 0)),
            pl.BlockSpec((1, hdim), lambda i: (0, 0)),
        ],
        out_specs=pl.BlockSpec((_BLK, hdim), lambda i: (i, 0)),
        out_shape=jax.ShapeDtypeStruct((n, hdim), jnp.float32),
    )(a, h0, pp, pp, prev, w, b.reshape(1, hdim))


def _layer0(h0, w, b, *, beta):
    n, hdim = h0.shape

    def bdy(h0_ref, w_ref, b_ref, o_ref):
        hi = h0_ref[...]
        x = ((1.0 - beta) * hi
             + beta * jnp.dot(hi, w_ref[...], preferred_element_type=jnp.float32)
             + b_ref[...])
        o_ref[...] = jnp.maximum(x, 0.0)

    return pl.pallas_call(
        bdy,
        grid=(n // _BLK,),
        in_specs=[
            pl.BlockSpec((_BLK, hdim), lambda i: (i, 0)),
            pl.BlockSpec((hdim, hdim), lambda i: (0, 0)),
            pl.BlockSpec((1, hdim), lambda i: (0, 0)),
        ],
        out_specs=pl.BlockSpec((_BLK, hdim), lambda i: (i, 0)),
        out_shape=jax.ShapeDtypeStruct((n, hdim), jnp.float32),
    )(h0, w, b.reshape(1, hdim))


def _final(x, w, b):
    n, hdim = x.shape
    c = w.shape[1]

    def bdy(x_ref, w_ref, b_ref, o_ref):
        t = jnp.maximum(x_ref[...], 0.0)
        y = (jnp.dot(t, w_ref[...], preferred_element_type=jnp.float32)
             + b_ref[...])
        m = jnp.max(y, axis=1, keepdims=True)
        lse = m + jnp.log(jnp.sum(jnp.exp(y - m), axis=1, keepdims=True))
        o_ref[...] = y - lse

    return pl.pallas_call(
        bdy,
        grid=(n // _BLK,),
        in_specs=[
            pl.BlockSpec((_BLK, hdim), lambda i: (i, 0)),
            pl.BlockSpec((hdim, c), lambda i: (0, 0)),
            pl.BlockSpec((1, c), lambda i: (0, 0)),
        ],
        out_specs=pl.BlockSpec((_BLK, c), lambda i: (i, 0)),
        out_shape=jax.ShapeDtypeStruct((n, c), jnp.float32),
    )(x, w, b.reshape(1, c))


def kernel(features, edge_index, norm_A, W_fc0, b_fc0, conv_W, conv_b,
           W_fc1, b_fc1, alpha_params):
    n = features.shape[0]
    e = norm_A.shape[0]
    hdim = W_fc0.shape[1]
    lnum = conv_W.shape[0] - 1

    wpw = -(-e // (_WIN * _NW))
    wpw = (wpw + _NBUF - 1) // _NBUF * _NBUF
    wpw = max(wpw, 2 * _NBUF)
    epad = wpw * _NW * _WIN
    pad = epad - e
    src2 = jnp.pad(edge_index[0], (0, pad)).reshape(epad // _WIN, _WIN)
    dst2 = jnp.pad(edge_index[1], (0, pad)).reshape(epad // _WIN, _WIN)
    norm2 = jnp.pad(norm_A, (0, pad)).reshape(epad // _WIN, _WIN)
    zer = jnp.zeros(((-(-n // _NS) + 7) // 8 * 8, hdim), jnp.float32)
    prop = _make_prop(n, wpw, hdim)

    h0 = _fc0(features, W_fc0, b_fc0)
    x = _layer0(h0, conv_W[0], conv_b[0],
                beta=math.log(_LAMDA / 1.0 + 1.0))
    prev = h0
    last = x
    for i in range(1, lnum + 1):
        pp = prop(last, src2, dst2, norm2, zer)
        a = alpha_params[lnum - i].reshape(1)
        beta = math.log(_LAMDA / (i + 1) + 1.0)
        xi = _layer(a, h0, pp, prev, conv_W[i], conv_b[i],
                    beta=beta, tmul=1.0 if i == 1 else 2.0,
                    pmul=0.0 if i == 1 else 1.0,
                    dorelu=i < lnum - 1)
        prev = last
        last = xi
    return _final(last, W_fc1, b_fc1)
